# baseline (speedup 1.0000x reference)
import jax
import jax.numpy as jnp
from jax.experimental import pallas as pl
from jax.experimental.pallas import tpu as pltpu
from jax.experimental.pallas import tpu_sc as plsc

IN_CH = 64
LATENT = 32
K = 8192
T = 1024
TH = T // 2
KB = 4096
NK = K // KB
_PREC = jax.lax.Precision.HIGHEST
_INT_MAX = 2**31 - 1


def _enc_argmin_body(x_ref, wc_ref, be_ref, emb_ref, ze_ref, idx_ref, tbl_ref, z_s, bv_s, bi_s):
    j = pl.program_id(0)

    @pl.when(j == 0)
    def _():
        x = x_ref[...]
        zc1 = jnp.zeros((IN_CH, 1), jnp.float32)
        zc2 = jnp.zeros((IN_CH, 2), jnp.float32)
        x4f = jnp.concatenate([
            jnp.concatenate([zc1, x[:, :-1]], axis=1),
            x,
            jnp.concatenate([x[:, 1:], zc1], axis=1),
            jnp.concatenate([x[:, 2:], zc2], axis=1),
        ], axis=0)
        zf = jax.lax.dot_general(wc_ref[...], x4f, (((1,), (0,)), ((), ())),
                                 precision=jax.lax.Precision.DEFAULT,
                                 preferred_element_type=jnp.float32)
        ri = jax.lax.broadcasted_iota(jnp.int32, (T, TH), 0)
        ci = jax.lax.broadcasted_iota(jnp.int32, (T, TH), 1)
        dsel = jnp.where(ri == 2 * ci, 1.0, 0.0)
        z = jax.lax.dot_general(zf, dsel, (((1,), (0,)), ((), ())),
                                precision=_PREC, preferred_element_type=jnp.float32)
        z = z + be_ref[...]
        z_s[...] = z
        ze_ref[...] = z

    e = emb_ref[...]
    tbl_ref[...] = jnp.concatenate([e, jnp.zeros((KB, 128 - LATENT), jnp.float32)], axis=1)
    g = jax.lax.dot_general(e, z_s[...], (((1,), (0,)), ((), ())),
                            precision=_PREC, preferred_element_type=jnp.float32)
    en = jnp.sum(e * e, axis=1, keepdims=True)
    scores = en - 2.0 * g
    lmin = jnp.min(scores, axis=0, keepdims=True)
    rowi = jax.lax.broadcasted_iota(jnp.int32, scores.shape, 0) + j * KB
    lidx = jnp.min(jnp.where(scores == lmin, rowi, _INT_MAX), axis=0, keepdims=True)

    @pl.when(j == 0)
    def _():
        bv_s[...] = lmin
        bi_s[...] = lidx

    @pl.when(j > 0)
    def _():
        better = lmin < bv_s[...]
        bv_s[...] = jnp.where(better, lmin, bv_s[...])
        bi_s[...] = jnp.where(better, lidx, bi_s[...])

    @pl.when(j == NK - 1)
    def _():
        idx_ref[...] = bi_s[...]


_NC = 2
_NS = 16
_NW = _NC * _NS
_BPW = TH // _NW


def _sc_gather_body(tbl_hbm, idx_hbm, out_hbm, idx_v, rows_v, sem):
    wid = jax.lax.axis_index("s") * _NC + jax.lax.axis_index("c")
    base = wid * _BPW
    pltpu.sync_copy(idx_hbm.at[pl.ds(base, _BPW)], idx_v)
    pltpu.async_copy(tbl_hbm.at[idx_v], rows_v, sem).wait()
    pltpu.sync_copy(rows_v, out_hbm.at[pl.ds(base, _BPW)])


def _sc_gather(tbl, idx_flat):
    mesh = plsc.VectorSubcoreMesh(core_axis_name="c", subcore_axis_name="s")
    return pl.kernel(
        _sc_gather_body,
        out_type=jax.ShapeDtypeStruct((TH, 128), jnp.float32),
        mesh=mesh,
        scratch_types=[
            pltpu.VMEM((_BPW,), jnp.int32),
            pltpu.VMEM((_BPW, 128), jnp.float32),
            pltpu.SemaphoreType.DMA,
        ],
    )(tbl, idx_flat)


def _decode_body(rows_ref, wd_ref, bd_ref, zq_ref, ev_ref, od_ref):
    zq = jnp.transpose(rows_ref[:, :LATENT], (1, 0))
    zq_ref[...] = zq
    r = jax.lax.dot_general(wd_ref[...], zq, (((1,), (0,)), ((), ())),
                            precision=_PREC, preferred_element_type=jnp.float32)
    r0 = r[0:64, :]
    r1 = r[64:128, :]
    r2 = r[128:192, :]
    r3 = r[192:256, :]
    zpad = jnp.zeros((64, 1), jnp.float32)
    shr = jnp.concatenate([zpad, r0[:, :-1]], axis=1)
    shl = jnp.concatenate([r3[:, 1:], zpad], axis=1)
    ev_ref[...] = shr + r2 + bd_ref[...]
    od_ref[...] = r1 + shl + bd_ref[...]


def kernel(x, W_enc, b_enc, emb, W_dec, b_dec):
    Wcat = jnp.transpose(W_enc, (0, 2, 1)).reshape(LATENT, 4 * IN_CH)
    Wd2 = jnp.concatenate([W_dec[:, :, 0], W_dec[:, :, 1],
                           W_dec[:, :, 2], W_dec[:, :, 3]], axis=0)

    z_e, idx2, tbl = pl.pallas_call(
        _enc_argmin_body,
        grid=(NK,),
        in_specs=[
            pl.BlockSpec((IN_CH, T), lambda j: (0, 0)),
            pl.BlockSpec((LATENT, 4 * IN_CH), lambda j: (0, 0)),
            pl.BlockSpec((LATENT, 1), lambda j: (0, 0)),
            pl.BlockSpec((KB, LATENT), lambda j: (j, 0)),
        ],
        out_specs=[
            pl.BlockSpec((LATENT, TH), lambda j: (0, 0)),
            pl.BlockSpec((1, TH), lambda j: (0, 0)),
            pl.BlockSpec((KB, 128), lambda j: (j, 0)),
        ],
        out_shape=[
            jax.ShapeDtypeStruct((LATENT, TH), jnp.float32),
            jax.ShapeDtypeStruct((1, TH), jnp.int32),
            jax.ShapeDtypeStruct((K, 128), jnp.float32),
        ],
        scratch_shapes=[
            pltpu.VMEM((LATENT, TH), jnp.float32),
            pltpu.VMEM((1, TH), jnp.float32),
            pltpu.VMEM((1, TH), jnp.int32),
        ],
    )(x, Wcat, b_enc, emb)

    indices = idx2.reshape(TH)
    rows128 = _sc_gather(tbl, indices)

    z_q, ev, od = pl.pallas_call(
        _decode_body,
        in_specs=[
            pl.BlockSpec((TH, 128), lambda: (0, 0)),
            pl.BlockSpec((4 * IN_CH, LATENT), lambda: (0, 0)),
            pl.BlockSpec((IN_CH, 1), lambda: (0, 0)),
        ],
        out_specs=[
            pl.BlockSpec((LATENT, TH), lambda: (0, 0)),
            pl.BlockSpec((IN_CH, TH), lambda: (0, 0)),
            pl.BlockSpec((IN_CH, TH), lambda: (0, 0)),
        ],
        out_shape=[
            jax.ShapeDtypeStruct((LATENT, TH), jnp.float32),
            jax.ShapeDtypeStruct((IN_CH, TH), jnp.float32),
            jax.ShapeDtypeStruct((IN_CH, TH), jnp.float32),
        ],
    )(rows128, Wd2, b_dec)

    x_recon = jnp.stack([ev, od], axis=-1).reshape(IN_CH, T)
    return (x_recon, z_e, z_q, indices)


# speedup vs baseline: 1.0158x; 1.0158x over previous
import jax
import jax.numpy as jnp
from jax.experimental import pallas as pl
from jax.experimental.pallas import tpu as pltpu
from jax.experimental.pallas import tpu_sc as plsc

IN_CH = 64
LATENT = 32
K = 8192
T = 1024
TH = T // 2
KB = 1024
NK = K // KB
_PREC = jax.lax.Precision.HIGHEST
_INT_MAX = 2**31 - 1


def _enc_argmin_body(x_ref, wc_ref, be_ref, emb_ref, ze_ref, idx_ref, tbl_ref, z_s, bv_s, bi_s):
    j = pl.program_id(0)

    @pl.when(j == 0)
    def _():
        x = x_ref[...]
        zc1 = jnp.zeros((IN_CH, 1), jnp.float32)
        zc2 = jnp.zeros((IN_CH, 2), jnp.float32)
        x4f = jnp.concatenate([
            jnp.concatenate([zc1, x[:, :-1]], axis=1),
            x,
            jnp.concatenate([x[:, 1:], zc1], axis=1),
            jnp.concatenate([x[:, 2:], zc2], axis=1),
        ], axis=0)
        zf = jax.lax.dot_general(wc_ref[...], x4f, (((1,), (0,)), ((), ())),
                                 precision=jax.lax.Precision.DEFAULT,
                                 preferred_element_type=jnp.float32)
        ri = jax.lax.broadcasted_iota(jnp.int32, (T, TH), 0)
        ci = jax.lax.broadcasted_iota(jnp.int32, (T, TH), 1)
        dsel = jnp.where(ri == 2 * ci, 1.0, 0.0)
        z = jax.lax.dot_general(zf, dsel, (((1,), (0,)), ((), ())),
                                precision=_PREC, preferred_element_type=jnp.float32)
        z = z + be_ref[...]
        z_s[...] = z
        ze_ref[...] = z

    e = emb_ref[...]
    tbl_ref[...] = jnp.concatenate([e, jnp.zeros((KB, 128 - LATENT), jnp.float32)], axis=1)
    g = jax.lax.dot_general(e, z_s[...], (((1,), (0,)), ((), ())),
                            precision=_PREC, preferred_element_type=jnp.float32)
    en = jnp.sum(e * e, axis=1, keepdims=True)
    scores = en - 2.0 * g
    lmin = jnp.min(scores, axis=0, keepdims=True)
    rowi = jax.lax.broadcasted_iota(jnp.int32, scores.shape, 0) + j * KB
    lidx = jnp.min(jnp.where(scores == lmin, rowi, _INT_MAX), axis=0, keepdims=True)

    @pl.when(j == 0)
    def _():
        bv_s[...] = lmin
        bi_s[...] = lidx

    @pl.when(j > 0)
    def _():
        better = lmin < bv_s[...]
        bv_s[...] = jnp.where(better, lmin, bv_s[...])
        bi_s[...] = jnp.where(better, lidx, bi_s[...])

    @pl.when(j == NK - 1)
    def _():
        idx_ref[...] = bi_s[...]


_NC = 2
_NS = 16
_NW = _NC * _NS
_BPW = TH // _NW


def _sc_gather_body(tbl_hbm, idx_hbm, out_hbm, idx_v, rows_v, sem):
    wid = jax.lax.axis_index("s") * _NC + jax.lax.axis_index("c")
    base = wid * _BPW
    pltpu.sync_copy(idx_hbm.at[pl.ds(base, _BPW)], idx_v)
    pltpu.async_copy(tbl_hbm.at[idx_v], rows_v, sem).wait()
    pltpu.sync_copy(rows_v, out_hbm.at[pl.ds(base, _BPW)])


def _sc_gather(tbl, idx_flat):
    mesh = plsc.VectorSubcoreMesh(core_axis_name="c", subcore_axis_name="s")
    return pl.kernel(
        _sc_gather_body,
        out_type=jax.ShapeDtypeStruct((TH, 128), jnp.float32),
        mesh=mesh,
        scratch_types=[
            pltpu.VMEM((_BPW,), jnp.int32),
            pltpu.VMEM((_BPW, 128), jnp.float32),
            pltpu.SemaphoreType.DMA,
        ],
    )(tbl, idx_flat)


def _decode_body(rows_ref, wd_ref, bd_ref, zq_ref, ev_ref, od_ref):
    zq = jnp.transpose(rows_ref[:, :LATENT], (1, 0))
    zq_ref[...] = zq
    r = jax.lax.dot_general(wd_ref[...], zq, (((1,), (0,)), ((), ())),
                            precision=_PREC, preferred_element_type=jnp.float32)
    r0 = r[0:64, :]
    r1 = r[64:128, :]
    r2 = r[128:192, :]
    r3 = r[192:256, :]
    zpad = jnp.zeros((64, 1), jnp.float32)
    shr = jnp.concatenate([zpad, r0[:, :-1]], axis=1)
    shl = jnp.concatenate([r3[:, 1:], zpad], axis=1)
    ev_ref[...] = shr + r2 + bd_ref[...]
    od_ref[...] = r1 + shl + bd_ref[...]


def kernel(x, W_enc, b_enc, emb, W_dec, b_dec):
    Wcat = jnp.transpose(W_enc, (0, 2, 1)).reshape(LATENT, 4 * IN_CH)
    Wd2 = jnp.concatenate([W_dec[:, :, 0], W_dec[:, :, 1],
                           W_dec[:, :, 2], W_dec[:, :, 3]], axis=0)

    z_e, idx2, tbl = pl.pallas_call(
        _enc_argmin_body,
        grid=(NK,),
        in_specs=[
            pl.BlockSpec((IN_CH, T), lambda j: (0, 0)),
            pl.BlockSpec((LATENT, 4 * IN_CH), lambda j: (0, 0)),
            pl.BlockSpec((LATENT, 1), lambda j: (0, 0)),
            pl.BlockSpec((KB, LATENT), lambda j: (j, 0)),
        ],
        out_specs=[
            pl.BlockSpec((LATENT, TH), lambda j: (0, 0)),
            pl.BlockSpec((1, TH), lambda j: (0, 0)),
            pl.BlockSpec((KB, 128), lambda j: (j, 0)),
        ],
        out_shape=[
            jax.ShapeDtypeStruct((LATENT, TH), jnp.float32),
            jax.ShapeDtypeStruct((1, TH), jnp.int32),
            jax.ShapeDtypeStruct((K, 128), jnp.float32),
        ],
        scratch_shapes=[
            pltpu.VMEM((LATENT, TH), jnp.float32),
            pltpu.VMEM((1, TH), jnp.float32),
            pltpu.VMEM((1, TH), jnp.int32),
        ],
    )(x, Wcat, b_enc, emb)

    indices = idx2.reshape(TH)
    rows128 = _sc_gather(tbl, indices)

    z_q, ev, od = pl.pallas_call(
        _decode_body,
        in_specs=[
            pl.BlockSpec((TH, 128), lambda: (0, 0)),
            pl.BlockSpec((4 * IN_CH, LATENT), lambda: (0, 0)),
            pl.BlockSpec((IN_CH, 1), lambda: (0, 0)),
        ],
        out_specs=[
            pl.BlockSpec((LATENT, TH), lambda: (0, 0)),
            pl.BlockSpec((IN_CH, TH), lambda: (0, 0)),
            pl.BlockSpec((IN_CH, TH), lambda: (0, 0)),
        ],
        out_shape=[
            jax.ShapeDtypeStruct((LATENT, TH), jnp.float32),
            jax.ShapeDtypeStruct((IN_CH, TH), jnp.float32),
            jax.ShapeDtypeStruct((IN_CH, TH), jnp.float32),
        ],
    )(rows128, Wd2, b_dec)

    x_recon = jnp.stack([ev, od], axis=-1).reshape(IN_CH, T)
    return (x_recon, z_e, z_q, indices)
